# Initial kernel scaffold; baseline (speedup 1.0000x reference)
import jax
import jax.numpy as jnp
from jax.experimental import pallas as pl


def kernel(vid0, vid1):
    raise NotImplementedError("write your pallas kernel here")


# speedup vs baseline: 481.6001x; 481.6001x over previous
import jax
import jax.numpy as jnp
from jax.experimental import pallas as pl
from jax.experimental.pallas import tpu as pltpu

_NEG = -3.0e38


def _nl_kernel(v0_ref, v1_ref, topd_ref, inds_ref, dists_ref):
    t = pl.program_id(0)
    d = pl.program_id(1)
    a = pl.program_id(2)

    is_dup = ((t == 0) & (d == 1)) | ((t == 2) & (d == 2))

    @pl.when(jnp.logical_not(is_dup))
    def _compute():
        rr = jax.lax.broadcasted_iota(jnp.int32, (32, 131), 0)
        yy = jax.lax.broadcasted_iota(jnp.int32, (32, 131), 1)
        diff = yy - 4 * rr
        sel = jnp.where((diff >= 0) & (diff <= 6), 1.0, 0.0).astype(jnp.float32)

        base = d * 64 + a * 8
        for b in range(8):
            m = v0_ref[0, 0, :131, :131] * v1_ref[0, 0, 0, :, b : b + 131]
            for c in range(1, 32):
                m = m + v0_ref[0, c, :131, :131] * v1_ref[0, 0, c, :, b : b + 131]
            dmat = jnp.dot(
                jnp.dot(sel, m, preferred_element_type=jnp.float32),
                sel.T,
                preferred_element_type=jnp.float32,
            )
            dists_ref[pl.ds(base + b, 1), :, :] = dmat[None]

    @pl.when(is_dup & (a == 0))
    def _copy():
        dists_ref[pl.ds(d * 64, 64), :, :] = dists_ref[pl.ds((d - 1) * 64, 64), :, :]

    @pl.when((d == 2) & (a == 7))
    def _topk():
        dv = dists_ref[:, :, :]
        iota = jax.lax.broadcasted_iota(jnp.int32, (192, 32, 32), 0)
        hq = jax.lax.broadcasted_iota(jnp.int32, (32, 32), 0)
        wq = jax.lax.broadcasted_iota(jnp.int32, (32, 32), 1)
        for k in range(7):
            mx = jnp.max(dv, axis=0)
            idx = jnp.min(jnp.where(dv == mx[None], iota, 192), axis=0)
            topd_ref[0, k] = mx
            dtv = idx // 64
            rem = idx - dtv * 64
            ia = rem // 8
            ib = rem - ia * 8
            inds_ref[0, k, 0] = jnp.clip(t + dtv - 1, 0, 2)
            inds_ref[0, k, 1] = jnp.clip(4 * hq + ia - 4, 0, 127)
            inds_ref[0, k, 2] = jnp.clip(4 * wq + ib - 4, 0, 127)
            if k < 6:
                dv = jnp.where(iota == idx[None], _NEG, dv)


@jax.jit
def kernel(vid0, vid1):
    v0p = jnp.pad(vid0[0], ((0, 0), (0, 0), (3, 3), (3, 3)), mode="reflect")
    v1p = jnp.pad(vid1[0], ((0, 0), (0, 0), (7, 3), (7, 3)), mode="reflect")
    v1a = jnp.stack([v1p[:, :, a : a + 131, :] for a in range(8)], axis=1)

    topd, inds = pl.pallas_call(
        _nl_kernel,
        grid=(3, 3, 8),
        in_specs=[
            pl.BlockSpec((1, 32, 134, 134), lambda t, d, a: (t, 0, 0, 0)),
            pl.BlockSpec(
                (1, 1, 32, 131, 138),
                lambda t, d, a: (jnp.clip(t + d - 1, 0, 2), a, 0, 0, 0),
            ),
        ],
        out_specs=[
            pl.BlockSpec((1, 7, 32, 32), lambda t, d, a: (t, 0, 0, 0)),
            pl.BlockSpec((1, 7, 3, 32, 32), lambda t, d, a: (t, 0, 0, 0, 0)),
        ],
        out_shape=[
            jax.ShapeDtypeStruct((3, 7, 32, 32), jnp.float32),
            jax.ShapeDtypeStruct((3, 7, 3, 32, 32), jnp.int32),
        ],
        scratch_shapes=[pltpu.VMEM((192, 32, 32), jnp.float32)],
    )(v0p, v1a)

    top_d = topd.transpose(0, 2, 3, 1).reshape(1, 1, 3072, 7)
    out_inds = inds.transpose(0, 3, 4, 1, 2).reshape(1, 1, 3072, 7, 3)
    return top_d, out_inds


# speedup vs baseline: 489.4338x; 1.0163x over previous
import jax
import jax.numpy as jnp
from jax.experimental import pallas as pl

_NEG = -3.0e38


def _dist_kernel(v0_ref, v1_ref, out_ref):
    rr = jax.lax.broadcasted_iota(jnp.int32, (32, 131), 0)
    yy = jax.lax.broadcasted_iota(jnp.int32, (32, 131), 1)
    diff = yy - 4 * rr
    sel = jnp.where((diff >= 0) & (diff <= 6), 1.0, 0.0).astype(jnp.float32)

    for b in range(8):
        m = v0_ref[0, 0, :131, :131] * v1_ref[0, 0, 0, :, b : b + 131]
        for c in range(1, 32):
            m = m + v0_ref[0, c, :131, :131] * v1_ref[0, 0, c, :, b : b + 131]
        dmat = jnp.dot(
            jnp.dot(sel, m, preferred_element_type=jnp.float32),
            sel.T,
            preferred_element_type=jnp.float32,
        )
        out_ref[0, b] = dmat


def _topk_kernel(d_ref, topd_ref, inds_ref):
    t = pl.program_id(0)
    dv = d_ref[0]
    iota = jax.lax.broadcasted_iota(jnp.int32, (192, 1024), 0)
    qi = jax.lax.broadcasted_iota(jnp.int32, (1, 1024), 1)
    hq = qi // 32
    wq = qi - 32 * hq
    for k in range(7):
        mx = jnp.max(dv, axis=0, keepdims=True)
        idx = jnp.min(jnp.where(dv == mx, iota, 192), axis=0, keepdims=True)
        topd_ref[0, pl.ds(k, 1), :] = mx
        dtv = idx // 64
        rem = idx - dtv * 64
        ia = rem // 8
        ib = rem - ia * 8
        inds_ref[0, pl.ds(k, 1), 0, :] = jnp.clip(t + dtv - 1, 0, 2)
        inds_ref[0, pl.ds(k, 1), 1, :] = jnp.clip(4 * hq + ia - 4, 0, 127)
        inds_ref[0, pl.ds(k, 1), 2, :] = jnp.clip(4 * wq + ib - 4, 0, 127)
        if k < 6:
            dv = jnp.where(iota == idx, _NEG, dv)


@jax.jit
def kernel(vid0, vid1):
    v0p = jnp.pad(vid0[0], ((0, 0), (0, 0), (3, 3), (3, 3)), mode="reflect")
    v1p = jnp.pad(vid1[0], ((0, 0), (0, 0), (7, 3), (7, 3)), mode="reflect")
    v1a = jnp.stack([v1p[:, :, a : a + 131, :] for a in range(8)], axis=1)

    dists_u = pl.pallas_call(
        _dist_kernel,
        grid=(7, 8),
        in_specs=[
            pl.BlockSpec((1, 32, 134, 134), lambda p, a: ((p + 1) // 3, 0, 0, 0)),
            pl.BlockSpec(
                (1, 1, 32, 131, 138),
                lambda p, a: (p - 2 * ((p + 1) // 3), a, 0, 0, 0),
            ),
        ],
        out_specs=pl.BlockSpec((1, 8, 32, 32), lambda p, a: (p, a, 0, 0)),
        out_shape=jax.ShapeDtypeStruct((7, 64, 32, 32), jnp.float32),
    )(v0p, v1a)

    dfull = dists_u[jnp.array([0, 0, 1, 2, 3, 4, 5, 6, 6])]
    dfull = dfull.reshape(3, 192, 1024)

    topd, inds = pl.pallas_call(
        _topk_kernel,
        grid=(3,),
        in_specs=[pl.BlockSpec((1, 192, 1024), lambda t: (t, 0, 0))],
        out_specs=[
            pl.BlockSpec((1, 7, 1024), lambda t: (t, 0, 0)),
            pl.BlockSpec((1, 7, 3, 1024), lambda t: (t, 0, 0, 0)),
        ],
        out_shape=[
            jax.ShapeDtypeStruct((3, 7, 1024), jnp.float32),
            jax.ShapeDtypeStruct((3, 7, 3, 1024), jnp.int32),
        ],
    )(dfull)

    top_d = topd.transpose(0, 2, 1).reshape(1, 1, 3072, 7)
    out_inds = inds.transpose(0, 3, 1, 2).reshape(1, 1, 3072, 7, 3)
    return top_d, out_inds


# speedup vs baseline: 502.5412x; 1.0268x over previous
import jax
import jax.numpy as jnp
from jax.experimental import pallas as pl
from jax.experimental.pallas import tpu as pltpu

_NEG = -3.0e38


def _dist_kernel(v0_ref, v1_ref, out_ref, s_ref):
    rr = jax.lax.broadcasted_iota(jnp.int32, (32, 131), 0)
    yy = jax.lax.broadcasted_iota(jnp.int32, (32, 131), 1)
    diff = yy - 4 * rr
    sel = jnp.where((diff >= 0) & (diff <= 6), 1.0, 0.0).astype(jnp.float32)

    for b in range(8):
        def build(c, carry):
            s_ref[pl.ds(c, 1)] = v1_ref[0, pl.ds(c, 1), :, b : b + 131]
            return carry

        jax.lax.fori_loop(0, 32, build, 0, unroll=4)

        for a in range(8):
            def body(c, m):
                return m + (
                    v0_ref[0, pl.ds(c, 1), :131, :131]
                    * s_ref[pl.ds(c, 1), a : a + 131, :]
                )

            m = jax.lax.fori_loop(
                0, 32, body, jnp.zeros((1, 131, 131), jnp.float32), unroll=4
            )
            dmat = jnp.dot(
                jnp.dot(sel, m[0], preferred_element_type=jnp.float32),
                sel.T,
                preferred_element_type=jnp.float32,
            )
            out_ref[0, a * 8 + b] = dmat


def _topk_kernel(d_ref, topd_ref, inds_ref):
    t = pl.program_id(0)
    dv = d_ref[0]
    iota = jax.lax.broadcasted_iota(jnp.int32, (192, 1024), 0)
    qi = jax.lax.broadcasted_iota(jnp.int32, (1, 1024), 1)
    hq = qi // 32
    wq = qi - 32 * hq
    for k in range(7):
        mx = jnp.max(dv, axis=0, keepdims=True)
        idx = jnp.min(jnp.where(dv == mx, iota, 192), axis=0, keepdims=True)
        topd_ref[0, pl.ds(k, 1), :] = mx
        dtv = idx // 64
        rem = idx - dtv * 64
        ia = rem // 8
        ib = rem - ia * 8
        inds_ref[0, pl.ds(k, 1), 0, :] = jnp.clip(t + dtv - 1, 0, 2)
        inds_ref[0, pl.ds(k, 1), 1, :] = jnp.clip(4 * hq + ia - 4, 0, 127)
        inds_ref[0, pl.ds(k, 1), 2, :] = jnp.clip(4 * wq + ib - 4, 0, 127)
        if k < 6:
            dv = jnp.where(iota == idx, _NEG, dv)


@jax.jit
def kernel(vid0, vid1):
    v0p = jnp.pad(vid0[0], ((0, 0), (0, 0), (3, 3), (3, 3)), mode="reflect")
    v1p = jnp.pad(vid1[0], ((0, 0), (0, 0), (7, 3), (7, 3)), mode="reflect")

    dists_u = pl.pallas_call(
        _dist_kernel,
        grid=(7,),
        in_specs=[
            pl.BlockSpec((1, 32, 134, 134), lambda p: ((p + 1) // 3, 0, 0, 0)),
            pl.BlockSpec((1, 32, 138, 138), lambda p: (p - 2 * ((p + 1) // 3), 0, 0, 0)),
        ],
        out_specs=pl.BlockSpec((1, 64, 32, 32), lambda p: (p, 0, 0, 0)),
        out_shape=jax.ShapeDtypeStruct((7, 64, 32, 32), jnp.float32),
        scratch_shapes=[pltpu.VMEM((32, 138, 131), jnp.float32)],
    )(v0p, v1p)

    dfull = dists_u[jnp.array([0, 0, 1, 2, 3, 4, 5, 6, 6])]
    dfull = dfull.reshape(3, 192, 1024)

    topd, inds = pl.pallas_call(
        _topk_kernel,
        grid=(3,),
        in_specs=[pl.BlockSpec((1, 192, 1024), lambda t: (t, 0, 0))],
        out_specs=[
            pl.BlockSpec((1, 7, 1024), lambda t: (t, 0, 0)),
            pl.BlockSpec((1, 7, 3, 1024), lambda t: (t, 0, 0, 0)),
        ],
        out_shape=[
            jax.ShapeDtypeStruct((3, 7, 1024), jnp.float32),
            jax.ShapeDtypeStruct((3, 7, 3, 1024), jnp.int32),
        ],
    )(dfull)

    top_d = topd.transpose(0, 2, 1).reshape(1, 1, 3072, 7)
    out_inds = inds.transpose(0, 3, 1, 2).reshape(1, 1, 3072, 7, 3)
    return top_d, out_inds
